# baseline (speedup 1.0000x reference)
import jax
import jax.numpy as jnp
from jax.experimental import pallas as pl
from jax.experimental.pallas import tpu as pltpu

E = 64
K = 8
R = 16
TEMP = 0.5
TILE = 512


def _h_kernel(x_ref, wt64_ref, b64_ref, h_ref, mx_ref):
    i = pl.program_id(0)
    h = jnp.dot(x_ref[...], wt64_ref[...], preferred_element_type=jnp.float32)
    h = h + b64_ref[...]
    h_ref[...] = h
    tmax = jnp.max(jnp.abs(h))

    @pl.when(i == 0)
    def _():
        mx_ref[0, 0] = tmax

    @pl.when(i != 0)
    def _():
        mx_ref[0, 0] = jnp.maximum(mx_ref[0, 0], tmax)


def _main_kernel(mx_ref, x_ref, h_ref, wt_ref, b_ref, a_ref, bm_ref,
                 lime_ref, o_ref):
    x = x_ref[...]
    h = h_ref[...]
    scale = jnp.maximum(mx_ref[0, 0], 1e-6)
    inv = (1.0 / TEMP) / scale
    ex = jnp.exp(h * inv)

    bits = jax.lax.bitcast_convert_type(ex, jnp.int32)
    iota = jax.lax.broadcasted_iota(jnp.int32, ex.shape, 1)
    cur = jax.lax.bitcast_convert_type((bits & -64) | (63 - iota),
                                       jnp.float32)
    for _ in range(K):
        mval = jnp.max(cur, axis=-1, keepdims=True)
        cur = jnp.where(cur == mval, 0.0, cur)

    w = jnp.where(cur == 0.0, ex, 0.0)
    s = jnp.sum(w, axis=-1, keepdims=True)
    wn = w / s
    p_mix = jnp.dot(wn, lime_ref[...], preferred_element_type=jnp.float32)

    x16 = x.astype(jnp.bfloat16)
    base = jnp.dot(x16, wt_ref[...], preferred_element_type=jnp.float32)
    base = base + b_ref[...]
    u = jnp.dot(x16, a_ref[...], preferred_element_type=jnp.float32)
    delta = jnp.dot(u * p_mix, bm_ref[...],
                    preferred_element_type=jnp.float32)
    o_ref[...] = base + delta


def kernel(x, W, b, A, Bm, LiMEs):
    Bb, T, D_in = x.shape
    D_out = W.shape[0]
    N = Bb * T
    x2 = x.reshape(N, D_in)
    Wt = W.T

    h, mx = pl.pallas_call(
        _h_kernel,
        grid=(N // TILE,),
        in_specs=[
            pl.BlockSpec((TILE, D_in), lambda i: (i, 0)),
            pl.BlockSpec((D_in, E), lambda i: (0, 0)),
            pl.BlockSpec((1, E), lambda i: (0, 0)),
        ],
        out_specs=[
            pl.BlockSpec((TILE, E), lambda i: (i, 0)),
            pl.BlockSpec(memory_space=pltpu.SMEM),
        ],
        out_shape=[
            jax.ShapeDtypeStruct((N, E), jnp.float32),
            jax.ShapeDtypeStruct((1, 1), jnp.float32),
        ],
    )(x2, Wt[:, :E], b[:E].reshape(1, E))

    out = pl.pallas_call(
        _main_kernel,
        grid=(N // TILE,),
        in_specs=[
            pl.BlockSpec(memory_space=pltpu.SMEM),
            pl.BlockSpec((TILE, D_in), lambda i: (i, 0)),
            pl.BlockSpec((TILE, E), lambda i: (i, 0)),
            pl.BlockSpec((D_in, D_out), lambda i: (0, 0)),
            pl.BlockSpec((1, D_out), lambda i: (0, 0)),
            pl.BlockSpec((D_in, R), lambda i: (0, 0)),
            pl.BlockSpec((R, D_out), lambda i: (0, 0)),
            pl.BlockSpec((E, R), lambda i: (0, 0)),
        ],
        out_specs=pl.BlockSpec((TILE, D_out), lambda i: (i, 0)),
        out_shape=jax.ShapeDtypeStruct((N, D_out), jnp.float32),
    )(mx, x2, h, Wt.astype(jnp.bfloat16), b.reshape(1, D_out),
      A.astype(jnp.bfloat16), Bm, LiMEs)

    return out.reshape(Bb, T, D_out)


# speedup vs baseline: 1.1442x; 1.1442x over previous
import jax
import jax.numpy as jnp
from jax.experimental import pallas as pl
from jax.experimental.pallas import tpu as pltpu

E = 64
K = 8
R = 16
TEMP = 0.5
TILE = 512


def _fused_kernel(x_ref, wt_ref, b_ref, a_ref, bm_ref, lime_ref, o_ref,
                  h_scr, mx_scr):
    ph = pl.program_id(0)
    i = pl.program_id(1)

    @pl.when(ph == 0)
    def _():
        h = jnp.dot(x_ref[...], wt_ref[:, :E],
                    preferred_element_type=jnp.float32)
        h = h + b_ref[:, :E]
        h_scr[i] = h
        tmax = jnp.max(jnp.abs(h))

        @pl.when(i == 0)
        def _():
            mx_scr[0, 0] = tmax

        @pl.when(i != 0)
        def _():
            mx_scr[0, 0] = jnp.maximum(mx_scr[0, 0], tmax)

    @pl.when(ph == 1)
    def _():
        x = x_ref[...]
        h = h_scr[i]
        scale = jnp.maximum(mx_scr[0, 0], 1e-6)
        inv = (1.0 / TEMP) / scale
        ex = jnp.exp(h * inv)

        bits = jax.lax.bitcast_convert_type(ex, jnp.int32)
        iota = jax.lax.broadcasted_iota(jnp.int32, ex.shape, 1)
        cur = jax.lax.bitcast_convert_type((bits & -64) | (63 - iota),
                                           jnp.float32)
        for _ in range(K):
            mval = jnp.max(cur, axis=-1, keepdims=True)
            cur = jnp.where(cur == mval, 0.0, cur)

        w = jnp.where(cur == 0.0, ex, 0.0)
        s = jnp.sum(w, axis=-1, keepdims=True)
        wn = w / s
        p_mix = jnp.dot(wn, lime_ref[...],
                        preferred_element_type=jnp.float32)

        base = jnp.dot(x, wt_ref[...], preferred_element_type=jnp.float32)
        base = base + b_ref[...]
        u = jnp.dot(x, a_ref[...], preferred_element_type=jnp.float32)
        delta = jnp.dot(u * p_mix, bm_ref[...],
                        preferred_element_type=jnp.float32)
        o_ref[...] = base + delta


def kernel(x, W, b, A, Bm, LiMEs):
    Bb, T, D_in = x.shape
    D_out = W.shape[0]
    N = Bb * T
    NT = N // TILE
    x2 = x.reshape(N, D_in)
    Wt = W.T

    out = pl.pallas_call(
        _fused_kernel,
        grid=(2, NT),
        in_specs=[
            pl.BlockSpec((TILE, D_in), lambda ph, i: (i, 0)),
            pl.BlockSpec((D_in, D_out), lambda ph, i: (0, 0)),
            pl.BlockSpec((1, D_out), lambda ph, i: (0, 0)),
            pl.BlockSpec((D_in, R), lambda ph, i: (0, 0)),
            pl.BlockSpec((R, D_out), lambda ph, i: (0, 0)),
            pl.BlockSpec((E, R), lambda ph, i: (0, 0)),
        ],
        out_specs=pl.BlockSpec((TILE, D_out),
                               lambda ph, i: (jnp.where(ph == 0, 0, i), 0)),
        out_shape=jax.ShapeDtypeStruct((N, D_out), jnp.float32),
        scratch_shapes=[
            pltpu.VMEM((NT, TILE, E), jnp.float32),
            pltpu.SMEM((1, 1), jnp.float32),
        ],
    )(x2, Wt, b.reshape(1, D_out), A, Bm, LiMEs)

    return out.reshape(Bb, T, D_out)


# speedup vs baseline: 1.3716x; 1.1987x over previous
import jax
import jax.numpy as jnp
from jax.experimental import pallas as pl
from jax.experimental.pallas import tpu as pltpu

E = 64
K = 8
R = 16
TEMP = 0.5
TILE = 1024


def _fused_kernel(x_ref, wt_ref, b_ref, a_ref, bm_ref, lime_ref, o_ref,
                  h_scr, mx_scr):
    ph = pl.program_id(0)
    i = pl.program_id(1)

    @pl.when(ph == 0)
    def _():
        h = jnp.dot(x_ref[...], wt_ref[:, :E],
                    preferred_element_type=jnp.float32)
        h = h + b_ref[:, :E]
        h_scr[i] = h
        tmax = jnp.max(jnp.abs(h))

        @pl.when(i == 0)
        def _():
            mx_scr[0, 0] = tmax

        @pl.when(i != 0)
        def _():
            mx_scr[0, 0] = jnp.maximum(mx_scr[0, 0], tmax)

    @pl.when(ph == 1)
    def _():
        x = x_ref[...]
        h = h_scr[i]
        scale = jnp.maximum(mx_scr[0, 0], 1e-6)
        inv = (1.0 / TEMP) / scale
        ex = jnp.exp(h * inv)

        bits = jax.lax.bitcast_convert_type(ex, jnp.int32)
        iota = jax.lax.broadcasted_iota(jnp.int32, ex.shape, 1)
        cur = jax.lax.bitcast_convert_type((bits & -64) | (63 - iota),
                                           jnp.float32)
        for _ in range(K):
            mval = jnp.max(cur, axis=-1, keepdims=True)
            cur = jnp.where(cur == mval, 0.0, cur)

        w = jnp.where(cur == 0.0, ex, 0.0)
        s = jnp.sum(w, axis=-1, keepdims=True)
        wn = w / s
        p_mix = jnp.dot(wn, lime_ref[...],
                        preferred_element_type=jnp.float32)

        base = jnp.dot(x, wt_ref[...], preferred_element_type=jnp.float32)
        base = base + b_ref[...]
        u = jnp.dot(x, a_ref[...], preferred_element_type=jnp.float32)
        delta = jnp.dot(u * p_mix, bm_ref[...],
                        preferred_element_type=jnp.float32)
        o_ref[...] = base + delta


def kernel(x, W, b, A, Bm, LiMEs):
    Bb, T, D_in = x.shape
    D_out = W.shape[0]
    N = Bb * T
    NT = N // TILE
    x2 = x.reshape(N, D_in)
    Wt = W.T

    out = pl.pallas_call(
        _fused_kernel,
        grid=(2, NT),
        in_specs=[
            pl.BlockSpec((TILE, D_in), lambda ph, i: (i, 0)),
            pl.BlockSpec((D_in, D_out), lambda ph, i: (0, 0)),
            pl.BlockSpec((1, D_out), lambda ph, i: (0, 0)),
            pl.BlockSpec((D_in, R), lambda ph, i: (0, 0)),
            pl.BlockSpec((R, D_out), lambda ph, i: (0, 0)),
            pl.BlockSpec((E, R), lambda ph, i: (0, 0)),
        ],
        out_specs=pl.BlockSpec((TILE, D_out),
                               lambda ph, i: (jnp.where(ph == 0, 0, i), 0)),
        out_shape=jax.ShapeDtypeStruct((N, D_out), jnp.float32),
        scratch_shapes=[
            pltpu.VMEM((NT, TILE, E), jnp.float32),
            pltpu.SMEM((1, 1), jnp.float32),
        ],
    )(x2, Wt, b.reshape(1, D_out), A, Bm, LiMEs)

    return out.reshape(Bb, T, D_out)


# speedup vs baseline: 1.4539x; 1.0600x over previous
import jax
import jax.numpy as jnp
from jax.experimental import pallas as pl
from jax.experimental.pallas import tpu as pltpu

E = 64
K = 8
R = 16
TEMP = 0.5
TILE = 2048


def _fused_kernel(x_ref, wt_ref, b_ref, a_ref, bm_ref, lime_ref, o_ref,
                  h_scr, mx_scr):
    ph = pl.program_id(0)
    i = pl.program_id(1)

    @pl.when(ph == 0)
    def _():
        h = jnp.dot(x_ref[...], wt_ref[:, :E],
                    preferred_element_type=jnp.float32)
        h = h + b_ref[:, :E]
        h_scr[i] = h
        tmax = jnp.max(jnp.abs(h))

        @pl.when(i == 0)
        def _():
            mx_scr[0, 0] = tmax

        @pl.when(i != 0)
        def _():
            mx_scr[0, 0] = jnp.maximum(mx_scr[0, 0], tmax)

    @pl.when(ph == 1)
    def _():
        x = x_ref[...]
        h = h_scr[i]
        scale = jnp.maximum(mx_scr[0, 0], 1e-6)
        inv = (1.0 / TEMP) / scale
        ex = jnp.exp(h * inv)

        bits = jax.lax.bitcast_convert_type(ex, jnp.int32)
        iota = jax.lax.broadcasted_iota(jnp.int32, ex.shape, 1)
        cur = jax.lax.bitcast_convert_type((bits & -64) | (63 - iota),
                                           jnp.float32)
        for _ in range(K):
            mval = jnp.max(cur, axis=-1, keepdims=True)
            cur = jnp.where(cur == mval, 0.0, cur)

        w = jnp.where(cur == 0.0, ex, 0.0)
        s = jnp.sum(w, axis=-1, keepdims=True)
        wn = w / s
        p_mix = jnp.dot(wn, lime_ref[...],
                        preferred_element_type=jnp.float32)

        base = jnp.dot(x, wt_ref[...], preferred_element_type=jnp.float32)
        base = base + b_ref[...]
        u = jnp.dot(x, a_ref[...], preferred_element_type=jnp.float32)
        delta = jnp.dot(u * p_mix, bm_ref[...],
                        preferred_element_type=jnp.float32)
        o_ref[...] = base + delta


def kernel(x, W, b, A, Bm, LiMEs):
    Bb, T, D_in = x.shape
    D_out = W.shape[0]
    N = Bb * T
    NT = N // TILE
    x2 = x.reshape(N, D_in)
    Wt = W.T

    out = pl.pallas_call(
        _fused_kernel,
        grid=(2, NT),
        in_specs=[
            pl.BlockSpec((TILE, D_in), lambda ph, i: (i, 0)),
            pl.BlockSpec((D_in, D_out), lambda ph, i: (0, 0)),
            pl.BlockSpec((1, D_out), lambda ph, i: (0, 0)),
            pl.BlockSpec((D_in, R), lambda ph, i: (0, 0)),
            pl.BlockSpec((R, D_out), lambda ph, i: (0, 0)),
            pl.BlockSpec((E, R), lambda ph, i: (0, 0)),
        ],
        out_specs=pl.BlockSpec((TILE, D_out),
                               lambda ph, i: (jnp.where(ph == 0, 0, i), 0)),
        out_shape=jax.ShapeDtypeStruct((N, D_out), jnp.float32),
        scratch_shapes=[
            pltpu.VMEM((NT, TILE, E), jnp.float32),
            pltpu.SMEM((1, 1), jnp.float32),
        ],
        compiler_params=pltpu.CompilerParams(
            vmem_limit_bytes=100 * 1024 * 1024),
    )(x2, Wt, b.reshape(1, D_out), A, Bm, LiMEs)

    return out.reshape(Bb, T, D_out)


# speedup vs baseline: 1.4977x; 1.0301x over previous
import jax
import jax.numpy as jnp
from jax.experimental import pallas as pl
from jax.experimental.pallas import tpu as pltpu

E = 64
K = 8
R = 16
TEMP = 0.5
TILE = 2048


def _fused_kernel(x_ref, wt_ref, b_ref, a_ref, bm_ref, lime_ref, o_ref,
                  h_scr, mx_scr):
    ph = pl.program_id(0)
    i = pl.program_id(1)

    @pl.when(ph == 0)
    def _():
        h = jax.lax.dot_general(
            x_ref[...], wt_ref[:E, :], (((1,), (1,)), ((), ())),
            preferred_element_type=jnp.float32)
        h = h + b_ref[:, :E]
        h_scr[i] = h
        tmax = jnp.max(jnp.abs(h))

        @pl.when(i == 0)
        def _():
            mx_scr[0, 0] = tmax

        @pl.when(i != 0)
        def _():
            mx_scr[0, 0] = jnp.maximum(mx_scr[0, 0], tmax)

    @pl.when(ph == 1)
    def _():
        x = x_ref[...]
        h = h_scr[i]
        scale = jnp.maximum(mx_scr[0, 0], 1e-6)
        inv = (1.0 / TEMP) / scale
        ex = jnp.exp(h * inv)

        bits = jax.lax.bitcast_convert_type(ex, jnp.int32)
        iota = jax.lax.broadcasted_iota(jnp.int32, ex.shape, 1)
        cur = jax.lax.bitcast_convert_type((bits & -64) | (63 - iota),
                                           jnp.float32)
        for _ in range(K):
            mval = jnp.max(cur, axis=-1, keepdims=True)
            cur = jnp.where(cur == mval, 0.0, cur)

        w = jnp.where(cur == 0.0, ex, 0.0)
        s = jnp.sum(w, axis=-1, keepdims=True)
        wn = w / s
        p_mix = jnp.dot(wn, lime_ref[...],
                        preferred_element_type=jnp.float32)

        base = jax.lax.dot_general(
            x, wt_ref[...], (((1,), (1,)), ((), ())),
            preferred_element_type=jnp.float32)
        base = base + b_ref[...]
        u = jnp.dot(x, a_ref[...], preferred_element_type=jnp.float32)
        delta = jnp.dot(u * p_mix, bm_ref[...],
                        preferred_element_type=jnp.float32)
        o_ref[...] = base + delta


def kernel(x, W, b, A, Bm, LiMEs):
    Bb, T, D_in = x.shape
    D_out = W.shape[0]
    N = Bb * T
    NT = N // TILE
    x2 = x.reshape(N, D_in)

    out = pl.pallas_call(
        _fused_kernel,
        grid=(2, NT),
        in_specs=[
            pl.BlockSpec((TILE, D_in), lambda ph, i: (i, 0)),
            pl.BlockSpec((D_in, D_out), lambda ph, i: (0, 0)),
            pl.BlockSpec((1, D_out), lambda ph, i: (0, 0)),
            pl.BlockSpec((D_in, R), lambda ph, i: (0, 0)),
            pl.BlockSpec((R, D_out), lambda ph, i: (0, 0)),
            pl.BlockSpec((E, R), lambda ph, i: (0, 0)),
        ],
        out_specs=pl.BlockSpec((TILE, D_out),
                               lambda ph, i: (jnp.where(ph == 0, 0, i), 0)),
        out_shape=jax.ShapeDtypeStruct((N, D_out), jnp.float32),
        scratch_shapes=[
            pltpu.VMEM((NT, TILE, E), jnp.float32),
            pltpu.SMEM((1, 1), jnp.float32),
        ],
        compiler_params=pltpu.CompilerParams(
            vmem_limit_bytes=100 * 1024 * 1024),
    )(x2, W, b.reshape(1, D_out), A, Bm, LiMEs)

    return out.reshape(Bb, T, D_out)
